# baseline (speedup 1.0000x reference)
import dataclasses
import functools

import jax
import jax.numpy as jnp
from jax import lax
from jax.experimental import pallas as pl
from jax.experimental.pallas import tpu as pltpu
from jax.experimental.pallas import tpu_sc as plsc

_MARGIN = 2.0
_B = 16384
_HID = 32
_NW = 32
_PER_W = _B // _NW
_CHUNK = 128
_NCHUNK = _PER_W // _CHUNK


def _tec_body(ph_hbm, pt_hbm, pr_hbm, nh_hbm, nt_hbm, nr_hbm,
              ent_hbm, rel_hbm, nrm_hbm, out_hbm,
              iph, ipt, ipr, inh, int_, inr,
              bph, bpt, bpr, bpn, bnh, bnt, bnr, bnn,
              stage, sem):
    wid = lax.axis_index("s") * 2 + lax.axis_index("c")
    base = wid * _PER_W

    pltpu.sync_copy(ph_hbm.at[pl.ds(base, _PER_W)], iph)
    pltpu.sync_copy(pt_hbm.at[pl.ds(base, _PER_W)], ipt)
    pltpu.sync_copy(pr_hbm.at[pl.ds(base, _PER_W)], ipr)
    pltpu.sync_copy(nh_hbm.at[pl.ds(base, _PER_W)], inh)
    pltpu.sync_copy(nt_hbm.at[pl.ds(base, _PER_W)], int_)
    pltpu.sync_copy(nr_hbm.at[pl.ds(base, _PER_W)], inr)

    acc0 = jnp.float32(0.0)

    def row_compute(i, acc):
        ph0 = bph[i, pl.ds(0, 16)]
        ph1 = bph[i, pl.ds(16, 16)]
        pt0 = bpt[i, pl.ds(0, 16)]
        pt1 = bpt[i, pl.ds(16, 16)]
        pr0 = bpr[i, pl.ds(0, 16)]
        pr1 = bpr[i, pl.ds(16, 16)]
        pn0 = bpn[i, pl.ds(0, 16)]
        pn1 = bpn[i, pl.ds(16, 16)]
        nh0 = bnh[i, pl.ds(0, 16)]
        nh1 = bnh[i, pl.ds(16, 16)]
        nt0 = bnt[i, pl.ds(0, 16)]
        nt1 = bnt[i, pl.ds(16, 16)]
        nr0 = bnr[i, pl.ds(0, 16)]
        nr1 = bnr[i, pl.ds(16, 16)]
        nn0 = bnn[i, pl.ds(0, 16)]
        nn1 = bnn[i, pl.ds(16, 16)]

        pd0 = ph0 - pt0
        pd1 = ph1 - pt1
        pdot = jnp.sum(pd0 * pn0 + pd1 * pn1)
        ps0 = pd0 + pr0 - pdot * pn0
        ps1 = pd1 + pr1 - pdot * pn1
        p_score = jnp.sum(jnp.abs(ps0) + jnp.abs(ps1))

        nd0 = nh0 - nt0
        nd1 = nh1 - nt1
        ndot = jnp.sum(nd0 * nn0 + nd1 * nn1)
        ns0 = nd0 + nr0 - ndot * nn0
        ns1 = nd1 + nr1 - ndot * nn1
        n_score = jnp.sum(jnp.abs(ns0) + jnp.abs(ns1))

        return acc + jnp.maximum(p_score - n_score + _MARGIN, 0.0)

    acc = acc0
    for k in range(_NCHUNK):
        sl = pl.ds(k * _CHUNK, _CHUNK)
        cps = [
            pltpu.async_copy(ent_hbm.at[iph.at[sl]], bph, sem),
            pltpu.async_copy(ent_hbm.at[ipt.at[sl]], bpt, sem),
            pltpu.async_copy(rel_hbm.at[ipr.at[sl]], bpr, sem),
            pltpu.async_copy(nrm_hbm.at[ipr.at[sl]], bpn, sem),
            pltpu.async_copy(ent_hbm.at[inh.at[sl]], bnh, sem),
            pltpu.async_copy(ent_hbm.at[int_.at[sl]], bnt, sem),
            pltpu.async_copy(rel_hbm.at[inr.at[sl]], bnr, sem),
            pltpu.async_copy(nrm_hbm.at[inr.at[sl]], bnn, sem),
        ]
        for c in cps:
            c.wait()
        acc = lax.fori_loop(0, _CHUNK, row_compute, acc)

    lane = lax.iota(jnp.int32, 16)
    stage[...] = jnp.where(lane == 0, acc, jnp.float32(0.0))
    pltpu.sync_copy(stage, out_hbm.at[wid])


@jax.jit
def _transh_loss_partials(p_h, p_t, p_r, n_h, n_t, n_r,
                          ent_emb, rel_emb, norm_vec):
    mesh = plsc.VectorSubcoreMesh(core_axis_name="c", subcore_axis_name="s")
    cp = pltpu.CompilerParams(use_tc_tiling_on_sc=False)
    if "needs_layout_passes" in pltpu.CompilerParams.__dataclass_fields__:
        cp = dataclasses.replace(cp, needs_layout_passes=False)
    run = pl.kernel(
        _tec_body,
        out_type=jax.ShapeDtypeStruct((_NW, 16), jnp.float32),
        mesh=mesh,
        compiler_params=cp,
        scratch_types=[
            pltpu.VMEM((_PER_W,), jnp.int32),
            pltpu.VMEM((_PER_W,), jnp.int32),
            pltpu.VMEM((_PER_W,), jnp.int32),
            pltpu.VMEM((_PER_W,), jnp.int32),
            pltpu.VMEM((_PER_W,), jnp.int32),
            pltpu.VMEM((_PER_W,), jnp.int32),
            pltpu.VMEM((_CHUNK, _HID), jnp.float32),
            pltpu.VMEM((_CHUNK, _HID), jnp.float32),
            pltpu.VMEM((_CHUNK, _HID), jnp.float32),
            pltpu.VMEM((_CHUNK, _HID), jnp.float32),
            pltpu.VMEM((_CHUNK, _HID), jnp.float32),
            pltpu.VMEM((_CHUNK, _HID), jnp.float32),
            pltpu.VMEM((_CHUNK, _HID), jnp.float32),
            pltpu.VMEM((_CHUNK, _HID), jnp.float32),
            pltpu.VMEM((16,), jnp.float32),
            pltpu.SemaphoreType.DMA,
        ],
    )
    return run(p_h.astype(jnp.int32), p_t.astype(jnp.int32),
               p_r.astype(jnp.int32), n_h.astype(jnp.int32),
               n_t.astype(jnp.int32), n_r.astype(jnp.int32),
               ent_emb, rel_emb, norm_vec)


def kernel(p_h, p_t, p_r, n_h, n_t, n_r, ent_emb, rel_emb, norm_vec):
    partials = _transh_loss_partials(p_h, p_t, p_r, n_h, n_t, n_r,
                                     ent_emb, rel_emb, norm_vec)
    return jnp.sum(partials)
